# baseline (speedup 1.0000x reference)
import jax
import jax.numpy as jnp
from jax.experimental import pallas as pl
from jax.experimental.pallas import tpu as pltpu

E = 16
D_MODEL = 1024
D_EXPERT = 2048
S = 2048
T = 128
NC = S // T + E
P = NC * T
NF = 2
FB = D_EXPERT // NF


def _router_body(x_ref, wg_ref, eid_ref, cnt_ref, aux_ref):
    logits = jax.lax.dot_general(
        x_ref[...], wg_ref[...], (((1,), (1,)), ((), ())),
        preferred_element_type=jnp.float32)
    mx = jnp.max(logits, axis=1, keepdims=True)
    idx = jax.lax.broadcasted_iota(jnp.int32, logits.shape, 1)
    eid = jnp.min(jnp.where(logits >= mx, idx, E), axis=1)
    eid_ref[...] = eid
    onehot = (idx == eid[:, None]).astype(jnp.int32)
    cnt_ref[...] = jnp.sum(onehot, axis=0)
    probs = jax.nn.softmax(logits, axis=1)
    usage = jnp.mean(probs, axis=0)
    aux_ref[...] = jnp.sum((usage - 1.0 / E) ** 2).reshape(1, 1)


def _route_meta_body(eid_ref, cnt_ref, sorted_ref, poff_ref, off_ref):
    def offs(e, row):
        c = cnt_ref[e]
        poff_ref[e] = row
        off_ref[e] = row
        return row + ((c + T - 1) // T) * T
    jax.lax.fori_loop(0, E, offs, 0)

    def scatter(s, _):
        e = eid_ref[s]
        p = off_ref[e]
        sorted_ref[p] = s
        off_ref[e] = p + 1
        return 0
    jax.lax.fori_loop(0, S, scatter, 0)


def _moe_body(sid_ref, poff_ref, cnt_ref, sidv_ref, x_ref,
              wg_ref, wu_ref, wd_ref, out_ref, ys_ref):
    f = pl.program_id(0)
    e = pl.program_id(1)

    c = cnt_ref[e]
    base = poff_ref[e]
    nch = (c + T - 1) // T
    wgb = wg_ref[0].astype(jnp.bfloat16)
    wub = wu_ref[0].astype(jnp.bfloat16)
    wdb = wd_ref[0].astype(jnp.bfloat16)

    def chunk(ci, _):
        cbase = pl.multiple_of(base + ci * T, T)
        gc = base // T + ci
        sid_all = sidv_ref[...]
        col_iota = jax.lax.broadcasted_iota(jnp.int32, (T, P // T), 1)
        sid_col = jnp.sum(jnp.where(col_iota == gc, sid_all, 0),
                          axis=1, keepdims=True)
        tok_iota = jax.lax.broadcasted_iota(jnp.int32, (T, S), 1)
        disp = (tok_iota == sid_col).astype(jnp.bfloat16)
        xs = jax.lax.dot_general(disp, x_ref[...],
                                 (((1,), (0,)), ((), ())),
                                 preferred_element_type=jnp.float32
                                 ).astype(jnp.bfloat16)
        g = jax.lax.dot_general(xs, wgb, (((1,), (1,)), ((), ())),
                                preferred_element_type=jnp.float32)
        u = jax.lax.dot_general(xs, wub, (((1,), (1,)), ((), ())),
                                preferred_element_type=jnp.float32)
        h = ((g * jax.nn.sigmoid(g)) * u).astype(jnp.bfloat16)
        ys_ref[...] = jax.lax.dot_general(
            h, wdb, (((1,), (1,)), ((), ())),
            preferred_element_type=jnp.float32)

        valid = jnp.minimum(c - ci * T, T)

        @pl.when(f == 0)
        def _set():
            def sc(i, _):
                tok = sid_ref[cbase + i]
                out_ref[pl.ds(tok, 1), :] = ys_ref[pl.ds(i, 1), :]
                return 0
            jax.lax.fori_loop(0, valid, sc, 0)

        @pl.when(f != 0)
        def _add():
            def sc(i, _):
                tok = sid_ref[cbase + i]
                out_ref[pl.ds(tok, 1), :] = (out_ref[pl.ds(tok, 1), :]
                                             + ys_ref[pl.ds(i, 1), :])
                return 0
            jax.lax.fori_loop(0, valid, sc, 0)
        return 0

    jax.lax.fori_loop(0, nch, chunk, 0)


@jax.jit
def kernel(x, Wg, Wgate, Wup, Wdown):
    x2 = x.reshape(S, D_MODEL)

    eid, cnt, aux = pl.pallas_call(
        _router_body,
        out_shape=[
            jax.ShapeDtypeStruct((S,), jnp.int32),
            jax.ShapeDtypeStruct((E,), jnp.int32),
            jax.ShapeDtypeStruct((1, 1), jnp.float32),
        ],
    )(x2, Wg)

    sorted_ids, poff = pl.pallas_call(
        _route_meta_body,
        grid_spec=pltpu.PrefetchScalarGridSpec(
            num_scalar_prefetch=2,
            grid=(1,),
            in_specs=[],
            out_specs=[
                pl.BlockSpec(memory_space=pltpu.SMEM),
                pl.BlockSpec(memory_space=pltpu.SMEM),
            ],
            scratch_shapes=[
                pltpu.SMEM((E,), jnp.int32),
            ],
        ),
        out_shape=[
            jax.ShapeDtypeStruct((P,), jnp.int32),
            jax.ShapeDtypeStruct((E,), jnp.int32),
        ],
    )(eid, cnt)

    x2b = x2.astype(jnp.bfloat16)
    sid2 = sorted_ids.reshape(P // T, T).T

    out = pl.pallas_call(
        _moe_body,
        grid_spec=pltpu.PrefetchScalarGridSpec(
            num_scalar_prefetch=3,
            grid=(NF, E),
            in_specs=[
                pl.BlockSpec((T, P // T), lambda f, e, sid, po, cn: (0, 0)),
                pl.BlockSpec((S, D_MODEL), lambda f, e, sid, po, cn: (0, 0)),
                pl.BlockSpec((1, FB, D_MODEL),
                             lambda f, e, sid, po, cn: (e, f, 0)),
                pl.BlockSpec((1, FB, D_MODEL),
                             lambda f, e, sid, po, cn: (e, f, 0)),
                pl.BlockSpec((1, D_MODEL, FB),
                             lambda f, e, sid, po, cn: (e, 0, f)),
            ],
            out_specs=pl.BlockSpec((S, D_MODEL),
                                   lambda f, e, sid, po, cn: (0, 0)),
            scratch_shapes=[
                pltpu.VMEM((T, D_MODEL), jnp.float32),
            ],
        ),
        out_shape=jax.ShapeDtypeStruct((S, D_MODEL), jnp.float32),
    )(sorted_ids, poff, cnt, sid2, x2b, Wgate, Wup, Wdown)

    return out.reshape(x.shape), aux[0, 0]


# speedup vs baseline: 1.1408x; 1.1408x over previous
import jax
import jax.numpy as jnp
from jax.experimental import pallas as pl
from jax.experimental.pallas import tpu as pltpu

E = 16
D_MODEL = 1024
D_EXPERT = 2048
S = 2048
T = 128
NC = S // T + E
P = NC * T
NF = 2
FB = D_EXPERT // NF


def _router_body(x_ref, wg_ref, eid_ref, cnt_ref, aux_ref):
    logits = jax.lax.dot_general(
        x_ref[...], wg_ref[...], (((1,), (1,)), ((), ())),
        preferred_element_type=jnp.float32)
    mx = jnp.max(logits, axis=1, keepdims=True)
    idx = jax.lax.broadcasted_iota(jnp.int32, logits.shape, 1)
    eid = jnp.min(jnp.where(logits >= mx, idx, E), axis=1)
    eid_ref[...] = eid
    onehot = (idx == eid[:, None]).astype(jnp.int32)
    cnt_ref[...] = jnp.sum(onehot, axis=0)
    probs = jax.nn.softmax(logits, axis=1)
    usage = jnp.mean(probs, axis=0)
    aux_ref[...] = jnp.sum((usage - 1.0 / E) ** 2).reshape(1, 1)


def _route_meta_body(eid_ref, cnt_ref, sorted_ref, poff_ref, off_ref):
    def offs(e, row):
        c = cnt_ref[e]
        poff_ref[e] = row
        off_ref[e] = row
        return row + ((c + T - 1) // T) * T
    jax.lax.fori_loop(0, E, offs, 0)

    def scatter(s, _):
        e = eid_ref[s]
        p = off_ref[e]
        sorted_ref[p] = s
        off_ref[e] = p + 1
        return 0
    jax.lax.fori_loop(0, S, scatter, 0)


def _moe_body(sid_ref, poff_ref, cnt_ref, x_ref,
              wg_ref, wu_ref, wd_ref, out_ref, xs_ref, ys_ref):
    e = pl.program_id(0)
    f = pl.program_id(1)

    c = cnt_ref[e]
    base = poff_ref[e]
    nch = (c + T - 1) // T

    def chunk(ci, _):
        cbase = base + ci * T
        lbase = pl.multiple_of(ci * T, T)
        valid = jnp.minimum(c - ci * T, T)

        @pl.when(f == 0)
        def _gather():
            def ga(i, _):
                tok = sid_ref[cbase + i]
                xs_ref[pl.ds(lbase + i, 1), :] = x_ref[pl.ds(tok, 1), :]
                return 0
            jax.lax.fori_loop(0, valid, ga, 0)

        xs = xs_ref[pl.ds(lbase, T), :]
        g = jax.lax.dot_general(xs, wg_ref[0], (((1,), (1,)), ((), ())),
                                preferred_element_type=jnp.float32)
        u = jax.lax.dot_general(xs, wu_ref[0], (((1,), (1,)), ((), ())),
                                preferred_element_type=jnp.float32)
        h = (g * jax.nn.sigmoid(g)) * u
        ys_ref[...] = jax.lax.dot_general(
            h, wd_ref[0], (((1,), (1,)), ((), ())),
            preferred_element_type=jnp.float32)

        @pl.when(f == 0)
        def _set():
            def sc(i, _):
                tok = sid_ref[cbase + i]
                out_ref[pl.ds(tok, 1), :] = ys_ref[pl.ds(i, 1), :]
                return 0
            jax.lax.fori_loop(0, valid, sc, 0)

        @pl.when(f != 0)
        def _add():
            def sc(i, _):
                tok = sid_ref[cbase + i]
                out_ref[pl.ds(tok, 1), :] = (out_ref[pl.ds(tok, 1), :]
                                             + ys_ref[pl.ds(i, 1), :])
                return 0
            jax.lax.fori_loop(0, valid, sc, 0)
        return 0

    jax.lax.fori_loop(0, nch, chunk, 0)


@jax.jit
def kernel(x, Wg, Wgate, Wup, Wdown):
    x2 = x.reshape(S, D_MODEL)

    eid, cnt, aux = pl.pallas_call(
        _router_body,
        out_shape=[
            jax.ShapeDtypeStruct((S,), jnp.int32),
            jax.ShapeDtypeStruct((E,), jnp.int32),
            jax.ShapeDtypeStruct((1, 1), jnp.float32),
        ],
    )(x2, Wg)

    sorted_ids, poff = pl.pallas_call(
        _route_meta_body,
        grid_spec=pltpu.PrefetchScalarGridSpec(
            num_scalar_prefetch=2,
            grid=(1,),
            in_specs=[],
            out_specs=[
                pl.BlockSpec(memory_space=pltpu.SMEM),
                pl.BlockSpec(memory_space=pltpu.SMEM),
            ],
            scratch_shapes=[
                pltpu.SMEM((E,), jnp.int32),
            ],
        ),
        out_shape=[
            jax.ShapeDtypeStruct((P,), jnp.int32),
            jax.ShapeDtypeStruct((E,), jnp.int32),
        ],
    )(eid, cnt)

    out = pl.pallas_call(
        _moe_body,
        grid_spec=pltpu.PrefetchScalarGridSpec(
            num_scalar_prefetch=3,
            grid=(E, NF),
            in_specs=[
                pl.BlockSpec((S, D_MODEL), lambda e, f, sid, po, cn: (0, 0)),
                pl.BlockSpec((1, FB, D_MODEL),
                             lambda e, f, sid, po, cn: (e, f, 0)),
                pl.BlockSpec((1, FB, D_MODEL),
                             lambda e, f, sid, po, cn: (e, f, 0)),
                pl.BlockSpec((1, D_MODEL, FB),
                             lambda e, f, sid, po, cn: (e, 0, f)),
            ],
            out_specs=pl.BlockSpec((S, D_MODEL),
                                   lambda e, f, sid, po, cn: (0, 0)),
            scratch_shapes=[
                pltpu.VMEM((S, D_MODEL), jnp.float32),
                pltpu.VMEM((T, D_MODEL), jnp.float32),
            ],
        ),
        out_shape=jax.ShapeDtypeStruct((S, D_MODEL), jnp.float32),
    )(sorted_ids, poff, cnt, x2, Wgate, Wup, Wdown)

    return out.reshape(x.shape), aux[0, 0]
